# baseline (speedup 1.0000x reference)
import functools

import jax
import jax.numpy as jnp
from jax import lax
from jax.experimental import pallas as pl
from jax.experimental.pallas import tpu as pltpu
from jax.experimental.pallas import tpu_sc as plsc

N, E, DIN, DH, DE, NCLS, NL, NG = 10000, 320000, 128, 256, 16, 10, 3, 16

DHH = DH // 2
NSUB = 16
EPT = E // NSUB
CH = 80
NCHUNK = EPT // CH
NPAD = 10240
RPT = NPAD // NSUB
ZR = 80


def _ln(x, g, b, eps=1e-5):
    m = x.mean(-1, keepdims=True)
    v = ((x - m) ** 2).mean(-1, keepdims=True)
    return (x - m) / jnp.sqrt(v + eps) * g + b


def _bn(x, g, b, eps=1e-5):
    m = x.mean(0)
    v = ((x - m) ** 2).mean(0)
    return (x - m) / jnp.sqrt(v + eps) * g + b


def _edge_body(srch, dsth, ab, ex, acc, sidx, didx, abb, exb, msg, sem, sem2, sub):
    def chunk(c, _):
        base = sub * EPT + c * CH
        pltpu.sync_copy(srch.at[pl.ds(base, CH)], sidx)
        pltpu.sync_copy(dsth.at[pl.ds(base, CH)], didx)
        cp1 = pltpu.async_copy(ab.at[sidx], abb, sem)
        cp2 = pltpu.async_copy(ex.at[pl.ds(base, CH)], exb, sem2)
        cp1.wait()
        cp2.wait()

        def row(i, _):
            for j in range(DHH // 16):
                a = abb[i, pl.ds(j * 16, 16)]
                b = abb[i, pl.ds(DHH + j * 16, 16)]
                cde = exb[i, pl.ds(j * 16, 16)]
                z = b + cde
                s = 1.0 / (1.0 + jnp.exp(-z))
                msg[i, pl.ds(j * 16, 16)] = a * s
            return 0

        lax.fori_loop(0, CH, row, 0, unroll=2)
        pltpu.sync_copy(msg, acc.at[didx], add=True)
        return 0

    lax.fori_loop(0, NCHUNK, chunk, 0)


def _edge_sc(srch, dsth, ab0, ab1, ex0, ex1, agg0, agg1,
             acc, sidx, didx, abb, exb, msg, sem, sem2):
    cid = lax.axis_index("c")
    sub = lax.axis_index("s")

    def zrow(i, _):
        for j in range(DHH // 16):
            msg[i, pl.ds(j * 16, 16)] = jnp.zeros((16,), jnp.float32)
        return 0
    lax.fori_loop(0, ZR, zrow, 0)
    for j in range(RPT // ZR):
        pltpu.sync_copy(msg, acc.at[pl.ds(sub * RPT + j * ZR, ZR)])
    plsc.subcore_barrier()

    @pl.when(cid == 0)
    def _():
        _edge_body(srch, dsth, ab0, ex0, acc, sidx, didx, abb, exb, msg,
                   sem, sem2, sub)

    @pl.when(cid == 1)
    def _():
        _edge_body(srch, dsth, ab1, ex1, acc, sidx, didx, abb, exb, msg,
                   sem, sem2, sub)

    plsc.subcore_barrier()

    @pl.when(cid == 0)
    def _():
        pltpu.sync_copy(acc.at[pl.ds(sub * RPT, RPT)],
                        agg0.at[pl.ds(sub * RPT, RPT)])

    @pl.when(cid == 1)
    def _():
        pltpu.sync_copy(acc.at[pl.ds(sub * RPT, RPT)],
                        agg1.at[pl.ds(sub * RPT, RPT)])


_edge_call = pl.kernel(
    _edge_sc,
    out_type=(jax.ShapeDtypeStruct((NPAD, DHH), jnp.float32),
              jax.ShapeDtypeStruct((NPAD, DHH), jnp.float32)),
    mesh=plsc.VectorSubcoreMesh(core_axis_name="c", subcore_axis_name="s"),
    scratch_types=[
        pltpu.VMEM_SHARED((NPAD, DHH), jnp.float32),
        pltpu.VMEM((CH,), jnp.int32),
        pltpu.VMEM((CH,), jnp.int32),
        pltpu.VMEM((CH, 2 * DHH), jnp.float32),
        pltpu.VMEM((CH, DHH), jnp.float32),
        pltpu.VMEM((CH, DHH), jnp.float32),
        pltpu.SemaphoreType.DMA,
        pltpu.SemaphoreType.DMA,
    ],
)


def kernel(x, edge_index, edge_attr, batch, params):
    p = params
    src, dst = edge_index[0], edge_index[1]
    t = jax.nn.relu(edge_attr @ p['e2n_W'] + p['e2n_b'])
    t = _ln(t, p['e2n_g'], p['e2n_be'])
    nf = jnp.zeros((N, DIN), jnp.float32).at[dst].add(t).at[src].add(t)
    deg = jnp.zeros((N,), jnp.float32).at[src].add(1.0).at[dst].add(1.0)
    nf = nf / jnp.maximum(deg, 1.0)[:, None]
    h = (x + nf) @ p['emb_W'] + p['emb_b']
    for i in range(NL):
        Ax = h @ p['WA'][i] + p['bA'][i]
        Bx = h @ p['WB'][i] + p['bB'][i]
        Cx = h @ p['WC'][i] + p['bC'][i]
        Dx = h @ p['WD'][i] + p['bD'][i]
        Ex = edge_attr @ p['WE'][i] + p['bE'][i]
        ce0 = Ex[:, :DHH] + Cx[dst, :DHH]
        ce1 = Ex[:, DHH:] + Cx[dst, DHH:]
        ab0 = jnp.concatenate([Ax[:, :DHH], Bx[:, :DHH]], axis=1)
        ab1 = jnp.concatenate([Ax[:, DHH:], Bx[:, DHH:]], axis=1)
        agg0, agg1 = _edge_call(src, dst, ab0, ab1, ce0, ce1)
        agg = jnp.concatenate([agg0[:N], agg1[:N]], axis=1)
        h = jax.nn.relu(_bn(agg * jax.nn.sigmoid(Dx) + h, p['bn_g'][i], p['bn_b'][i]))
    d = jnp.abs(h[src] - h[dst])
    ep = jax.nn.relu(d @ p['dec_W1'] + p['dec_b1']) @ p['dec_W2'] + p['dec_b2']
    adj_pred = jax.nn.sigmoid(ep)[:, 0]
    gsum = jax.ops.segment_sum(h, batch, num_segments=NG)
    gcnt = jax.ops.segment_sum(jnp.ones((N,), jnp.float32), batch, num_segments=NG)
    gemb = gsum / jnp.maximum(gcnt, 1.0)[:, None]
    class_logits = jax.nn.relu(gemb @ p['cls_W1'] + p['cls_b1']) @ p['cls_W2'] + p['cls_b2']
    return (adj_pred, class_logits, h)


# speedup vs baseline: 1.0534x; 1.0534x over previous
import functools

import jax
import jax.numpy as jnp
from jax import lax
from jax.experimental import pallas as pl
from jax.experimental.pallas import tpu as pltpu
from jax.experimental.pallas import tpu_sc as plsc

N, E, DIN, DH, DE, NCLS, NL, NG = 10000, 320000, 128, 256, 16, 10, 3, 16

DHH = DH // 2
NSUB = 16
EPT = E // NSUB
CH = 80
NCHUNK = EPT // CH
NPAD = 10240
RPT = NPAD // NSUB
ZR = 80


def _ln(x, g, b, eps=1e-5):
    m = x.mean(-1, keepdims=True)
    v = ((x - m) ** 2).mean(-1, keepdims=True)
    return (x - m) / jnp.sqrt(v + eps) * g + b


def _bn(x, g, b, eps=1e-5):
    m = x.mean(0)
    v = ((x - m) ** 2).mean(0)
    return (x - m) / jnp.sqrt(v + eps) * g + b


def _edge_body(srch, dsth, ab, ex, acc, sidx, didx, abb, exb, msg, sem, sem2, sub):
    def chunk(c, _):
        base = sub * EPT + c * CH
        pltpu.sync_copy(srch.at[pl.ds(base, CH)], sidx)
        pltpu.sync_copy(dsth.at[pl.ds(base, CH)], didx)
        cp1 = pltpu.async_copy(ab.at[sidx], abb, sem)
        cp2 = pltpu.async_copy(ex.at[pl.ds(base, CH)], exb, sem2)
        cp1.wait()
        cp2.wait()

        def row(i, _):
            for j in range(DHH // 16):
                a = abb[i, pl.ds(j * 16, 16)]
                b = abb[i, pl.ds(DHH + j * 16, 16)]
                cde = exb[i, pl.ds(j * 16, 16)]
                z = b + cde
                s = 1.0 / (1.0 + jnp.exp(-z))
                msg[i, pl.ds(j * 16, 16)] = a * s
            return 0

        lax.fori_loop(0, CH, row, 0, unroll=2)
        pltpu.sync_copy(msg, acc.at[didx], add=True)
        return 0

    lax.fori_loop(0, NCHUNK, chunk, 0)


def _edge_sc(srch, dsth, ab0, ab1, ex0, ex1, agg0, agg1,
             acc, sidx, didx, abb, exb, msg, sem, sem2):
    cid = lax.axis_index("c")
    sub = lax.axis_index("s")

    def zrow(i, _):
        for j in range(DHH // 16):
            msg[i, pl.ds(j * 16, 16)] = jnp.zeros((16,), jnp.float32)
        return 0
    lax.fori_loop(0, ZR, zrow, 0)
    for j in range(RPT // ZR):
        pltpu.sync_copy(msg, acc.at[pl.ds(sub * RPT + j * ZR, ZR)])
    plsc.subcore_barrier()

    @pl.when(cid == 0)
    def _():
        _edge_body(srch, dsth, ab0, ex0, acc, sidx, didx, abb, exb, msg,
                   sem, sem2, sub)

    @pl.when(cid == 1)
    def _():
        _edge_body(srch, dsth, ab1, ex1, acc, sidx, didx, abb, exb, msg,
                   sem, sem2, sub)

    plsc.subcore_barrier()

    @pl.when(cid == 0)
    def _():
        pltpu.sync_copy(acc.at[pl.ds(sub * RPT, RPT)],
                        agg0.at[pl.ds(sub * RPT, RPT)])

    @pl.when(cid == 1)
    def _():
        pltpu.sync_copy(acc.at[pl.ds(sub * RPT, RPT)],
                        agg1.at[pl.ds(sub * RPT, RPT)])


_edge_call = pl.kernel(
    _edge_sc,
    out_type=(jax.ShapeDtypeStruct((NPAD, DHH), jnp.float32),
              jax.ShapeDtypeStruct((NPAD, DHH), jnp.float32)),
    mesh=plsc.VectorSubcoreMesh(core_axis_name="c", subcore_axis_name="s"),
    compiler_params=pltpu.CompilerParams(use_tc_tiling_on_sc=False),
    scratch_types=[
        pltpu.VMEM_SHARED((NPAD, DHH), jnp.float32),
        pltpu.VMEM((CH,), jnp.int32),
        pltpu.VMEM((CH,), jnp.int32),
        pltpu.VMEM((CH, 2 * DHH), jnp.float32),
        pltpu.VMEM((CH, DHH), jnp.float32),
        pltpu.VMEM((CH, DHH), jnp.float32),
        pltpu.SemaphoreType.DMA,
        pltpu.SemaphoreType.DMA,
    ],
)


def kernel(x, edge_index, edge_attr, batch, params):
    p = params
    src, dst = edge_index[0], edge_index[1]
    t = jax.nn.relu(edge_attr @ p['e2n_W'] + p['e2n_b'])
    t = _ln(t, p['e2n_g'], p['e2n_be'])
    nf = jnp.zeros((N, DIN), jnp.float32).at[dst].add(t).at[src].add(t)
    deg = jnp.zeros((N,), jnp.float32).at[src].add(1.0).at[dst].add(1.0)
    nf = nf / jnp.maximum(deg, 1.0)[:, None]
    h = (x + nf) @ p['emb_W'] + p['emb_b']
    for i in range(NL):
        Ax = h @ p['WA'][i] + p['bA'][i]
        Bx = h @ p['WB'][i] + p['bB'][i]
        Cx = h @ p['WC'][i] + p['bC'][i]
        Dx = h @ p['WD'][i] + p['bD'][i]
        Ex = edge_attr @ p['WE'][i] + p['bE'][i]
        ce0 = Ex[:, :DHH] + Cx[dst, :DHH]
        ce1 = Ex[:, DHH:] + Cx[dst, DHH:]
        ab0 = jnp.concatenate([Ax[:, :DHH], Bx[:, :DHH]], axis=1)
        ab1 = jnp.concatenate([Ax[:, DHH:], Bx[:, DHH:]], axis=1)
        agg0, agg1 = _edge_call(src, dst, ab0, ab1, ce0, ce1)
        agg = jnp.concatenate([agg0[:N], agg1[:N]], axis=1)
        h = jax.nn.relu(_bn(agg * jax.nn.sigmoid(Dx) + h, p['bn_g'][i], p['bn_b'][i]))
    d = jnp.abs(h[src] - h[dst])
    ep = jax.nn.relu(d @ p['dec_W1'] + p['dec_b1']) @ p['dec_W2'] + p['dec_b2']
    adj_pred = jax.nn.sigmoid(ep)[:, 0]
    gsum = jax.ops.segment_sum(h, batch, num_segments=NG)
    gcnt = jax.ops.segment_sum(jnp.ones((N,), jnp.float32), batch, num_segments=NG)
    gemb = gsum / jnp.maximum(gcnt, 1.0)[:, None]
    class_logits = jax.nn.relu(gemb @ p['cls_W1'] + p['cls_b1']) @ p['cls_W2'] + p['cls_b2']
    return (adj_pred, class_logits, h)
